# baseline (speedup 1.0000x reference)
import jax
import jax.numpy as jnp
from jax import lax
from jax.experimental import pallas as pl
from jax.experimental.pallas import tpu as pltpu
from jax.experimental.pallas import tpu_sc as plsc

NUM_COLS = 2048
NUM_ROWS = 16384
NC = 2
NS = 16
L = 16
NW = NC * NS
ROWS_PER_W = NUM_ROWS // NW
R = 8
BLK = R * NUM_COLS
NBLK = ROWS_PER_W // R
CHUNKS = NUM_COLS // L
JU = 8


def _body(in_hbm, perm_hbm, out_hbm, perm_v,
          in_v0, in_v1, out_v0, out_v1,
          sem_in0, sem_in1, sem_out0, sem_out1):
    in_bufs = (in_v0, in_v1)
    out_bufs = (out_v0, out_v1)
    sem_in = (sem_in0, sem_in1)
    sem_out = (sem_out0, sem_out1)

    wid = lax.axis_index("c") * NS + lax.axis_index("s")
    elem0 = wid * (ROWS_PER_W * NUM_COLS)
    pltpu.sync_copy(perm_hbm, perm_v)

    def in_desc(b, p):
        return pltpu.make_async_copy(
            in_hbm.at[pl.ds(elem0 + b * BLK, BLK)], in_bufs[p], sem_in[p])

    def out_desc(b, p):
        return pltpu.make_async_copy(
            out_bufs[p], out_hbm.at[pl.ds(elem0 + b * BLK, BLK)], sem_out[p])

    in_desc(0, 0).start()
    in_desc(1, 1).start()

    def pair(i, carry):
        for p in range(2):
            b = 2 * i + p
            in_desc(b, p).wait()

            @pl.when(i >= 1)
            def _():
                out_desc(b - 2, p).wait()

            in_v = in_bufs[p]
            out_v = out_bufs[p]

            @plsc.parallel_loop(0, CHUNKS, unroll=JU)
            def _(j):
                j0 = j * L
                idx = perm_v[pl.ds(j0, L)]
                for r in range(R):
                    vals = plsc.load_gather(in_v, [idx + (r * NUM_COLS)])
                    out_v[pl.ds(r * NUM_COLS + j0, L)] = vals
            out_desc(b, p).start()

            @pl.when(i < NBLK // 2 - 1)
            def _():
                in_desc(b + 2, p).start()
        return carry

    lax.fori_loop(0, NBLK // 2, pair, 0)
    out_desc(NBLK - 2, 0).wait()
    out_desc(NBLK - 1, 1).wait()


@jax.jit
def _shuffle(inputs_flat, perm_i32):
    mesh = plsc.VectorSubcoreMesh(core_axis_name="c", subcore_axis_name="s")
    return pl.kernel(
        _body,
        out_type=jax.ShapeDtypeStruct((NUM_ROWS * NUM_COLS,), jnp.float32),
        mesh=mesh,
        compiler_params=pltpu.CompilerParams(needs_layout_passes=False),
        scratch_types=[
            pltpu.VMEM((NUM_COLS,), jnp.int32),
            pltpu.VMEM((BLK,), jnp.float32),
            pltpu.VMEM((BLK,), jnp.float32),
            pltpu.VMEM((BLK,), jnp.float32),
            pltpu.VMEM((BLK,), jnp.float32),
            pltpu.SemaphoreType.DMA,
            pltpu.SemaphoreType.DMA,
            pltpu.SemaphoreType.DMA,
            pltpu.SemaphoreType.DMA,
        ],
    )(inputs_flat, perm_i32)


def kernel(inputs, perm):
    out_flat = _shuffle(inputs.reshape(-1), perm.astype(jnp.int32))
    out = out_flat.reshape(NUM_ROWS, NUM_COLS)
    logdet = jnp.zeros((inputs.shape[0], 1), dtype=inputs.dtype)
    return (out, logdet)


# speedup vs baseline: 2.9616x; 2.9616x over previous
import jax
import jax.numpy as jnp
from jax import lax
from jax.experimental import pallas as pl
from jax.experimental.pallas import tpu as pltpu
from jax.experimental.pallas import tpu_sc as plsc

NUM_COLS = 2048
NUM_ROWS = 16384
NC = 2
NS = 16
L = 16
NW = NC * NS
ROWS_PER_W = NUM_ROWS // NW
R = 8
BLK = R * NUM_COLS
NBLK = ROWS_PER_W // R
CHUNKS = NUM_COLS // L
JU = 8


def _body(in_hbm, perm_hbm, out_hbm, perm_v,
          in_v0, in_v1, out_v0, out_v1,
          sem_in0, sem_in1, sem_out0, sem_out1):
    in_bufs = (in_v0, in_v1)
    out_bufs = (out_v0, out_v1)
    sem_in = (sem_in0, sem_in1)
    sem_out = (sem_out0, sem_out1)

    wid = lax.axis_index("c") * NS + lax.axis_index("s")
    pltpu.sync_copy(perm_hbm, perm_v)

    def in_desc(b, p):
        return pltpu.make_async_copy(
            in_hbm.at[pl.ds((wid + NW * b) * R, R), :], in_bufs[p], sem_in[p])

    def out_desc(b, p):
        return pltpu.make_async_copy(
            out_bufs[p], out_hbm.at[pl.ds((wid + NW * b) * R, R), :], sem_out[p])

    in_desc(0, 0).start()
    in_desc(1, 1).start()

    def pair(i, carry):
        for p in range(2):
            b = 2 * i + p
            in_desc(b, p).wait()

            @pl.when(i >= 1)
            def _():
                out_desc(b - 2, p).wait()

            in_v = in_bufs[p]
            out_v = out_bufs[p]

            @plsc.parallel_loop(0, CHUNKS, unroll=JU)
            def _(j):
                j0 = j * L
                idx = perm_v[pl.ds(j0, L)]
                zeros = idx - idx
                for r in range(R):
                    vals = plsc.load_gather(in_v, [zeros + r, idx])
                    out_v[r, pl.ds(j0, L)] = vals
            out_desc(b, p).start()

            @pl.when(i < NBLK // 2 - 1)
            def _():
                in_desc(b + 2, p).start()
        return carry

    lax.fori_loop(0, NBLK // 2, pair, 0)
    out_desc(NBLK - 2, 0).wait()
    out_desc(NBLK - 1, 1).wait()


@jax.jit
def _shuffle(inputs, perm_i32):
    mesh = plsc.VectorSubcoreMesh(core_axis_name="c", subcore_axis_name="s")
    return pl.kernel(
        _body,
        out_type=jax.ShapeDtypeStruct((NUM_ROWS, NUM_COLS), jnp.float32),
        mesh=mesh,
        compiler_params=pltpu.CompilerParams(needs_layout_passes=False),
        scratch_types=[
            pltpu.VMEM((NUM_COLS,), jnp.int32),
            pltpu.VMEM((R, NUM_COLS), jnp.float32),
            pltpu.VMEM((R, NUM_COLS), jnp.float32),
            pltpu.VMEM((R, NUM_COLS), jnp.float32),
            pltpu.VMEM((R, NUM_COLS), jnp.float32),
            pltpu.SemaphoreType.DMA,
            pltpu.SemaphoreType.DMA,
            pltpu.SemaphoreType.DMA,
            pltpu.SemaphoreType.DMA,
        ],
    )(inputs, perm_i32)


def kernel(inputs, perm):
    out = _shuffle(inputs, perm.astype(jnp.int32))
    logdet = jnp.zeros((inputs.shape[0], 1), dtype=inputs.dtype)
    return (out, logdet)


# speedup vs baseline: 3.0334x; 1.0242x over previous
import jax
import jax.numpy as jnp
from jax import lax
from jax.experimental import pallas as pl
from jax.experimental.pallas import tpu as pltpu
from jax.experimental.pallas import tpu_sc as plsc

NUM_COLS = 2048
NUM_ROWS = 16384
NC = 2
NS = 16
L = 16
NW = NC * NS
ROWS_PER_W = NUM_ROWS // NW
R = 4
BLK = R * NUM_COLS
NBLK = ROWS_PER_W // R
CHUNKS = NUM_COLS // L
JU = 8


NDB = 4


def _body(in_hbm, perm_hbm, out_hbm, perm_v, *rest):
    in_bufs = rest[0:NDB]
    out_bufs = rest[NDB:2 * NDB]
    sem_in = rest[2 * NDB:3 * NDB]
    sem_out = rest[3 * NDB:4 * NDB]

    wid = lax.axis_index("c") * NS + lax.axis_index("s")
    pltpu.sync_copy(perm_hbm, perm_v)

    def in_desc(b, p):
        return pltpu.make_async_copy(
            in_hbm.at[pl.ds((wid + NW * b) * R, R), :], in_bufs[p], sem_in[p])

    def out_desc(b, p):
        return pltpu.make_async_copy(
            out_bufs[p], out_hbm.at[pl.ds((wid + NW * b) * R, R), :], sem_out[p])

    for p in range(NDB):
        in_desc(p, p).start()

    def pair(i, carry):
        for p in range(NDB):
            b = NDB * i + p
            in_desc(b, p).wait()

            @pl.when(i >= 1)
            def _():
                out_desc(b - NDB, p).wait()

            in_v = in_bufs[p]
            out_v = out_bufs[p]

            @plsc.parallel_loop(0, CHUNKS, unroll=JU)
            def _(j):
                j0 = j * L
                idx = perm_v[pl.ds(j0, L)]
                zeros = idx - idx
                for r in range(R):
                    vals = plsc.load_gather(in_v, [zeros + r, idx])
                    out_v[r, pl.ds(j0, L)] = vals
            out_desc(b, p).start()

            @pl.when(i < NBLK // NDB - 1)
            def _():
                in_desc(b + NDB, p).start()
        return carry

    lax.fori_loop(0, NBLK // NDB, pair, 0)
    for p in range(NDB):
        out_desc(NBLK - NDB + p, p).wait()


@jax.jit
def _shuffle(inputs, perm_i32):
    mesh = plsc.VectorSubcoreMesh(core_axis_name="c", subcore_axis_name="s")
    return pl.kernel(
        _body,
        out_type=jax.ShapeDtypeStruct((NUM_ROWS, NUM_COLS), jnp.float32),
        mesh=mesh,
        compiler_params=pltpu.CompilerParams(needs_layout_passes=False),
        scratch_types=[
            pltpu.VMEM((NUM_COLS,), jnp.int32),
            *[pltpu.VMEM((R, NUM_COLS), jnp.float32) for _ in range(8)],
            *[pltpu.SemaphoreType.DMA for _ in range(8)],
        ],
    )(inputs, perm_i32)


def kernel(inputs, perm):
    out = _shuffle(inputs, perm.astype(jnp.int32))
    logdet = jnp.zeros((inputs.shape[0], 1), dtype=inputs.dtype)
    return (out, logdet)
